# Initial kernel scaffold; baseline (speedup 1.0000x reference)
import functools

import jax
import jax.numpy as jnp
from jax import lax
from jax.experimental import pallas as pl
from jax.experimental.pallas import tpu as pltpu
from jax.experimental.pallas import tpu_sc as plsc


def _matmul_body(x_ref, w_ref, o_ref):
    o_ref[0, ...] = jnp.dot(
        x_ref[...], w_ref[...], preferred_element_type=jnp.float32
    )


def _support_halves(x, W):
    n, din = x.shape
    dout = W.shape[1]
    h = dout // 2
    rb = 1000
    return pl.pallas_call(
        _matmul_body,
        grid=(2, n // rb),
        in_specs=[
            pl.BlockSpec((rb, din), lambda c, r: (r, 0)),
            pl.BlockSpec((din, h), lambda c, r: (0, c)),
        ],
        out_specs=pl.BlockSpec((1, rb, h), lambda c, r: (c, r, 0)),
        out_shape=jax.ShapeDtypeStruct((2, n, h), jnp.float32),
    )(x, W)


_CH = 128
_NSUB = 16
_H = 128
_LANES = 16


def _spmm(sup2, src, dst, wgt, bias2, n):
    epad = src.shape[0]
    chunks_per_sub = epad // (_CH * _NSUB)
    rows_per_sub = n // _NSUB
    rows_blk = rows_per_sub // 5
    mesh = plsc.VectorSubcoreMesh(core_axis_name="c", subcore_axis_name="s")

    @functools.partial(
        pl.kernel,
        out_type=jax.ShapeDtypeStruct((2, n, _H), jnp.float32),
        mesh=mesh,
        scratch_types=[
            pltpu.VMEM_SHARED((n, _H), jnp.float32),
            pltpu.VMEM((_CH,), jnp.int32),
            pltpu.VMEM((_CH,), jnp.int32),
            pltpu.VMEM((_CH,), jnp.float32),
            pltpu.VMEM((_CH, _H), jnp.float32),
            pltpu.VMEM((_H,), jnp.float32),
            pltpu.VMEM((125, _H), jnp.float32),
            pltpu.SemaphoreType.DMA,
        ],
    )
    def spmm_kernel(
        sup_hbm,
        src_hbm,
        dst_hbm,
        wgt_hbm,
        bias_hbm,
        out_hbm,
        acc_sh,
        src_v,
        dst_v,
        w_v,
        rows_v,
        bias_v,
        tile_v,
        sem,
    ):
        c = lax.axis_index("c")
        s = lax.axis_index("s")
        rows_blk = 125

        pltpu.sync_copy(bias_hbm.at[c], bias_v)

        def fill_row(r):
            for f in range(_H // _LANES):
                sl = pl.ds(f * _LANES, _LANES)
                tile_v[r, sl] = bias_v[sl]

        pl.loop(0, rows_blk)(fill_row)

        row0 = s * rows_per_sub
        for k in range(rows_per_sub // rows_blk):
            pltpu.sync_copy(tile_v, acc_sh.at[pl.ds(row0 + k * rows_blk, rows_blk)])

        plsc.subcore_barrier()

        def chunk(j):
            base = (s * chunks_per_sub + j) * _CH
            pltpu.sync_copy(src_hbm.at[pl.ds(base, _CH)], src_v)
            pltpu.sync_copy(dst_hbm.at[pl.ds(base, _CH)], dst_v)
            pltpu.sync_copy(wgt_hbm.at[pl.ds(base, _CH)], w_v)
            pltpu.async_copy(sup_hbm.at[c].at[src_v], rows_v, sem).wait()

            def scale_group(g):
                wvec = w_v[pl.ds(g * _LANES, _LANES)]
                for i in range(_LANES):
                    wb = jnp.full((_LANES,), wvec[i], jnp.float32)
                    e = g * _LANES + i
                    for f in range(_H // _LANES):
                        sl = pl.ds(f * _LANES, _LANES)
                        rows_v[e, sl] = rows_v[e, sl] * wb

            pl.loop(0, _CH // _LANES)(scale_group)
            pltpu.sync_copy(rows_v, acc_sh.at[dst_v], add=True)

        pl.loop(0, chunks_per_sub)(chunk)

        plsc.subcore_barrier()

        for k in range(rows_per_sub // rows_blk):
            r = row0 + k * rows_blk
            pltpu.sync_copy(
                acc_sh.at[pl.ds(r, rows_blk)], out_hbm.at[c, pl.ds(r, rows_blk)]
            )

    return spmm_kernel(sup2, src, dst, wgt, bias2)


def kernel(x, edge_index, edge_weight, W, bias):
    n = x.shape[0]
    e = edge_index.shape[1]
    epad = ((e + _CH * _NSUB - 1) // (_CH * _NSUB)) * (_CH * _NSUB)
    pad = epad - e
    src = jnp.concatenate([edge_index[0], jnp.zeros((pad,), jnp.int32)])
    dst = jnp.concatenate([edge_index[1], jnp.zeros((pad,), jnp.int32)])
    wgt = jnp.concatenate([edge_weight, jnp.zeros((pad,), jnp.float32)])
    bias2 = bias.reshape(2, _H)

    sup2 = _support_halves(x, W)
    out2 = _spmm(sup2, src, dst, wgt, bias2, n)
    return out2.transpose(1, 0, 2).reshape(n, 2 * _H)


# speedup vs baseline: 2.7040x; 2.7040x over previous
import functools

import jax
import jax.numpy as jnp
from jax import lax
from jax.experimental import pallas as pl
from jax.experimental.pallas import tpu as pltpu
from jax.experimental.pallas import tpu_sc as plsc


def _matmul_body(x_ref, w_ref, o_ref):
    o_ref[0, ...] = jnp.dot(
        x_ref[...], w_ref[...], preferred_element_type=jnp.float32
    )


def _support_halves(x, W):
    n, din = x.shape
    dout = W.shape[1]
    h = dout // 2
    rb = 1000
    return pl.pallas_call(
        _matmul_body,
        grid=(2, n // rb),
        in_specs=[
            pl.BlockSpec((rb, din), lambda c, r: (r, 0)),
            pl.BlockSpec((din, h), lambda c, r: (0, c)),
        ],
        out_specs=pl.BlockSpec((1, rb, h), lambda c, r: (c, r, 0)),
        out_shape=jax.ShapeDtypeStruct((2, n, h), jnp.float32),
    )(x, W)


_CH = 128
_NSUB = 16
_H = 128
_LANES = 16


def _spmm(sup2, src, dst, wgt, bias2, n_pad):
    epad = src.shape[0]
    chunks_per_sub = epad // (_CH * _NSUB)
    rows_per_sub = n_pad // _NSUB
    rows_blk = 128
    mesh = plsc.VectorSubcoreMesh(core_axis_name="c", subcore_axis_name="s")

    @functools.partial(
        pl.kernel,
        out_type=jax.ShapeDtypeStruct((2, n_pad, _H), jnp.float32),
        mesh=mesh,
        scratch_types=[
            pltpu.VMEM_SHARED((n_pad, _H), jnp.float32),
            pltpu.VMEM((_CH,), jnp.int32),
            pltpu.VMEM((_CH,), jnp.int32),
            pltpu.VMEM((_CH,), jnp.float32),
            pltpu.VMEM((_CH, _H), jnp.float32),
            pltpu.VMEM((_H,), jnp.float32),
            pltpu.VMEM((128, _H), jnp.float32),
            pltpu.SemaphoreType.DMA,
        ],
    )
    def spmm_kernel(
        sup_hbm,
        src_hbm,
        dst_hbm,
        wgt_hbm,
        bias_hbm,
        out_hbm,
        acc_sh,
        src_v,
        dst_v,
        w_v,
        rows_v,
        bias_v,
        tile_v,
        sem,
    ):
        c = lax.axis_index("c")
        s = lax.axis_index("s")

        pltpu.sync_copy(bias_hbm.at[c], bias_v)

        def fill_row(r):
            for f in range(_H // _LANES):
                sl = pl.ds(f * _LANES, _LANES)
                tile_v[r, sl] = bias_v[sl]

        pl.loop(0, rows_blk)(fill_row)

        row0 = s * rows_per_sub
        for k in range(rows_per_sub // rows_blk):
            pltpu.sync_copy(tile_v, acc_sh.at[pl.ds(row0 + k * rows_blk, rows_blk)])

        plsc.subcore_barrier()

        def chunk(j):
            base = (s * chunks_per_sub + j) * _CH
            pltpu.sync_copy(src_hbm.at[pl.ds(base, _CH)], src_v)
            pltpu.sync_copy(dst_hbm.at[pl.ds(base, _CH)], dst_v)
            pltpu.sync_copy(wgt_hbm.at[pl.ds(base, _CH)], w_v)
            pltpu.async_copy(sup_hbm.at[c].at[src_v], rows_v, sem).wait()

            def scale_group(g):
                wvec = w_v[pl.ds(g * _LANES, _LANES)]
                for i in range(_LANES):
                    wb = jnp.full((_LANES,), wvec[i], jnp.float32)
                    e = g * _LANES + i
                    for f in range(_H // _LANES):
                        sl = pl.ds(f * _LANES, _LANES)
                        rows_v[e, sl] = rows_v[e, sl] * wb

            pl.loop(0, _CH // _LANES)(scale_group)
            pltpu.sync_copy(rows_v, acc_sh.at[dst_v], add=True)

        pl.loop(0, chunks_per_sub)(chunk)

        plsc.subcore_barrier()

        for k in range(rows_per_sub // rows_blk):
            r = row0 + k * rows_blk
            pltpu.sync_copy(
                acc_sh.at[pl.ds(r, rows_blk)], out_hbm.at[c, pl.ds(r, rows_blk)]
            )

    return spmm_kernel(sup2, src, dst, wgt, bias2)


def kernel(x, edge_index, edge_weight, W, bias):
    n = x.shape[0]
    e = edge_index.shape[1]
    epad = ((e + _CH * _NSUB - 1) // (_CH * _NSUB)) * (_CH * _NSUB)
    pad = epad - e
    src = jnp.concatenate([edge_index[0], jnp.zeros((pad,), jnp.int32)])
    dst = jnp.concatenate([edge_index[1], jnp.zeros((pad,), jnp.int32)])
    wgt = jnp.concatenate([edge_weight, jnp.zeros((pad,), jnp.float32)])
    bias2 = bias.reshape(2, _H)

    n_pad = ((n + _NSUB * 128 - 1) // (_NSUB * 128)) * (_NSUB * 128)
    sup2 = _support_halves(x, W)
    out2 = _spmm(sup2, src, dst, wgt, bias2, n_pad)
    return out2[:, :n].transpose(1, 0, 2).reshape(n, 2 * _H)
